# speedup vs baseline: 3.5061x; 3.5061x over previous
import functools

import jax
import jax.numpy as jnp
from jax import lax
from jax.experimental import pallas as pl
from jax.experimental.pallas import tpu as pltpu
from jax.experimental.pallas import tpu_sc as plsc

VOCAB = 100000
D = 128
BATCH = 4096
SEQ = 200
N = BATCH * SEQ
EPS = 1e-12

NC = 2
NS = 16
NW = NC * NS
PER_W = N // NW
C = 128
NCHUNK = PER_W // C


def _sc_gather_body(ids_hbm, table_hbm, out_hbm, idx_v, rows_v, sem):
    wid = lax.axis_index("s") * NC + lax.axis_index("c")
    base = wid * PER_W

    @pl.loop(0, NCHUNK)
    def _chunk(g):
        off = base + g * C
        pltpu.sync_copy(ids_hbm.at[pl.ds(off, C)], idx_v)
        pltpu.async_copy(table_hbm.at[idx_v], rows_v, sem).wait()
        pltpu.sync_copy(rows_v, out_hbm.at[pl.ds(off, C)])


_sc_gather = functools.partial(
    pl.kernel,
    out_type=jax.ShapeDtypeStruct((N, D), jnp.float32),
    mesh=plsc.VectorSubcoreMesh(core_axis_name="c", subcore_axis_name="s"),
    scratch_types=[
        pltpu.VMEM((C,), jnp.int32),
        pltpu.VMEM((C, D), jnp.float32),
        pltpu.SemaphoreType.DMA,
    ],
)(_sc_gather_body)


def _ln_body(x_ref, g_ref, b_ref, o_ref):
    x = x_ref[...]
    mean = jnp.mean(x, axis=1, keepdims=True)
    cent = x - mean
    var = jnp.mean(cent * cent, axis=1, keepdims=True)
    o_ref[...] = cent * lax.rsqrt(var + EPS) * g_ref[...] + b_ref[...]


_LN_ROWS = 2048


def _tc_layernorm(x, gamma, beta):
    return pl.pallas_call(
        _ln_body,
        grid=(N // _LN_ROWS,),
        in_specs=[
            pl.BlockSpec((_LN_ROWS, D), lambda i: (i, 0)),
            pl.BlockSpec((1, D), lambda i: (0, 0)),
            pl.BlockSpec((1, D), lambda i: (0, 0)),
        ],
        out_specs=pl.BlockSpec((_LN_ROWS, D), lambda i: (i, 0)),
        out_shape=jax.ShapeDtypeStruct((N, D), jnp.float32),
    )(x, gamma.reshape(1, D), beta.reshape(1, D))


def kernel(input_ids, table, gamma, beta):
    ids = input_ids.reshape(-1).astype(jnp.int32)
    rows = _sc_gather(ids, table)
    out = _tc_layernorm(rows, gamma, beta)
    return out.reshape(BATCH, SEQ, D)


# speedup vs baseline: 4.6030x; 1.3129x over previous
import functools

import jax
import jax.numpy as jnp
from jax import lax
from jax.experimental import pallas as pl
from jax.experimental.pallas import tpu as pltpu
from jax.experimental.pallas import tpu_sc as plsc

VOCAB = 100000
D = 128
BATCH = 4096
SEQ = 200
N = BATCH * SEQ
EPS = 1e-12

NC = 2
NS = 16
NW = NC * NS
PER_W = N // NW
C = 128
NCHUNK = PER_W // C


NBUF = 4


def _sc_gather_body(ids_hbm, table_hbm, out_hbm, idx_all, bufs, sems_in, sems_out):
    wid = lax.axis_index("s") * NC + lax.axis_index("c")
    base = wid * PER_W

    pltpu.sync_copy(ids_hbm.at[pl.ds(base, PER_W)], idx_all)

    def fire_gather(g, s):
        pltpu.async_copy(
            table_hbm.at[idx_all.at[pl.ds(g * C, C)]], bufs[s], sems_in[s])

    def wait_gather(g, s):
        pltpu.make_async_copy(
            table_hbm.at[idx_all.at[pl.ds(g * C, C)]], bufs[s], sems_in[s]).wait()

    def fire_write(g, s):
        pltpu.async_copy(bufs[s], out_hbm.at[pl.ds(base + g * C, C)], sems_out[s])

    def wait_write(g, s):
        pltpu.make_async_copy(
            bufs[s], out_hbm.at[pl.ds(base + g * C, C)], sems_out[s]).wait()

    fire_gather(0, 0)
    fire_gather(1, 1)
    for g in (0, 1):
        s = g % NBUF
        wait_gather(g, s)
        fire_write(g, s)
        fire_gather(g + 2, (g + 2) % NBUF)

    @pl.loop(2, NCHUNK - 2, step=NBUF)
    def _outer(g0):
        for b in range(NBUF):
            g = g0 + b
            s = (2 + b) % NBUF
            wait_gather(g, s)
            fire_write(g, s)
            wait_write(g - 2, (s + 2) % NBUF)
            fire_gather(g + 2, (s + 2) % NBUF)

    for g in (NCHUNK - 2, NCHUNK - 1):
        s = g % NBUF
        wait_gather(g, s)
        fire_write(g, s)
        wait_write(g - 2, (g - 2) % NBUF)
    for g in (NCHUNK - 2, NCHUNK - 1):
        wait_write(g, g % NBUF)


_sc_gather = functools.partial(
    pl.kernel,
    out_type=jax.ShapeDtypeStruct((N, D), jnp.float32),
    mesh=plsc.VectorSubcoreMesh(core_axis_name="c", subcore_axis_name="s"),
    scratch_types=[
        pltpu.VMEM((PER_W,), jnp.int32),
        [pltpu.VMEM((C, D), jnp.float32) for _ in range(NBUF)],
        [pltpu.SemaphoreType.DMA for _ in range(NBUF)],
        [pltpu.SemaphoreType.DMA for _ in range(NBUF)],
    ],
)(_sc_gather_body)


def _ln_body(x_ref, g_ref, b_ref, o_ref):
    x = x_ref[...]
    mean = jnp.mean(x, axis=1, keepdims=True)
    cent = x - mean
    var = jnp.mean(cent * cent, axis=1, keepdims=True)
    o_ref[...] = cent * lax.rsqrt(var + EPS) * g_ref[...] + b_ref[...]


_LN_ROWS = 2048


def _tc_layernorm(x, gamma, beta):
    return pl.pallas_call(
        _ln_body,
        grid=(N // _LN_ROWS,),
        in_specs=[
            pl.BlockSpec((_LN_ROWS, D), lambda i: (i, 0)),
            pl.BlockSpec((1, D), lambda i: (0, 0)),
            pl.BlockSpec((1, D), lambda i: (0, 0)),
        ],
        out_specs=pl.BlockSpec((_LN_ROWS, D), lambda i: (i, 0)),
        out_shape=jax.ShapeDtypeStruct((N, D), jnp.float32),
    )(x, gamma.reshape(1, D), beta.reshape(1, D))


def kernel(input_ids, table, gamma, beta):
    ids = input_ids.reshape(-1).astype(jnp.int32)
    rows = _sc_gather(ids, table)
    out = _tc_layernorm(rows, gamma, beta)
    return out.reshape(BATCH, SEQ, D)
